# baseline (speedup 1.0000x reference)
import functools

import jax
import jax.numpy as jnp
from jax import lax
from jax.experimental import pallas as pl
from jax.experimental.pallas import tpu as pltpu
from jax.experimental.pallas import tpu_sc as plsc

N_NODES = 10000
N_EDGES = 10000
N_INC = 320000
D = 128
H = 64
W = 72
NC = 2
NS = 16
K = 128
INC_PER_TILE = N_INC // NS
NB = -(-INC_PER_TILE // K)
PAD = NB * K - INC_PER_TILE
TRASH = N_NODES
ACC_ROWS = N_NODES + 8
RING = 4
ROWS_PER_TILE = N_NODES // NS
ECH = 125


def _stage(table_ref, idx_hbm, acc, iring, rows, isem, gsem, ssem, c, s):
    def fire_idx(j):
        pltpu.async_copy(idx_hbm.at[c, s, j], iring.at[lax.rem(j, RING)],
                         isem.at[lax.rem(j, RING)])

    def wait_idx(j):
        pltpu.make_async_copy(idx_hbm.at[c, s, j],
                              iring.at[lax.rem(j, RING)],
                              isem.at[lax.rem(j, RING)]).wait()

    def fire_gather(j):
        slot = lax.rem(j, RING)
        pltpu.async_copy(table_ref.at[iring.at[slot, 0]], rows.at[slot],
                         gsem.at[slot])

    def wait_gather(j):
        slot = lax.rem(j, RING)
        pltpu.make_async_copy(table_ref.at[iring.at[slot, 0]],
                              rows.at[slot], gsem.at[slot]).wait()

    def fire_scatter(j):
        slot = lax.rem(j, RING)
        pltpu.async_copy(rows.at[slot], acc.at[iring.at[slot, 1]],
                         ssem.at[slot], add=True)

    def wait_scatter(j):
        slot = lax.rem(j, RING)
        pltpu.make_async_copy(rows.at[slot], acc.at[iring.at[slot, 1]],
                              ssem.at[slot]).wait()

    for t in range(RING):
        fire_idx(t)
    for g in range(2):
        wait_idx(g)
        fire_gather(g)

    def step(j, _):
        @pl.when(jnp.logical_and(j >= 1, j - 1 + RING < NB))
        def _():
            wait_scatter(j - 1)
            fire_idx(j - 1 + RING)

        @pl.when(j + 2 < NB)
        def _():
            wait_idx(j + 2)
            fire_gather(j + 2)

        wait_gather(j)
        fire_scatter(j)
        return 0

    lax.fori_loop(0, NB, step, 0)

    def drain(r, _):
        wait_scatter(r)
        return 0

    lax.fori_loop(NB - RING, NB, drain, 0)


def _normalize(acc, rows, s, write_back):
    lane = lax.iota(jnp.int32, 16)
    ebuf = rows.at[0, pl.ds(0, ECH)]
    for ch in range(ROWS_PER_TILE // ECH):
        base = s * ROWS_PER_TILE + ch * ECH
        pltpu.sync_copy(acc.at[pl.ds(base, ECH)], ebuf)

        def erow(i, _):
            tailv = rows[0, i, pl.ds(W - 16, 16)]
            cnt = tailv[8]
            cntv = jnp.full((16,), cnt, jnp.float32)
            invv = jnp.where(cntv > 0.0, 1.0 / cntv, jnp.float32(0.0))
            tail = jnp.where(lane < 8, tailv * invv,
                             jnp.where(lane == 8, jnp.float32(1.0),
                                       jnp.float32(0.0)))
            for q in range(H // 16):
                rows[0, i, pl.ds(q * 16, 16)] = (
                    rows[0, i, pl.ds(q * 16, 16)] * invv)
            rows[0, i, pl.ds(W - 16, 16)] = tail
            return 0

        lax.fori_loop(0, ECH, erow, 0)
        write_back(base, ebuf)


def _sc_body(table_hbm, idx1_hbm, idx2_hbm, zrows_hbm, out_hbm,
             acc1, acc2, iring, rows, isem, gsem, ssem):
    c = lax.axis_index("c")
    s = lax.axis_index("s")

    zbuf = rows.at[0, pl.ds(0, ECH)]
    pltpu.sync_copy(zrows_hbm, zbuf)
    for ch in range(ROWS_PER_TILE // ECH):
        base = s * ROWS_PER_TILE + ch * ECH
        pltpu.sync_copy(zbuf, acc1.at[pl.ds(base, ECH)])
        pltpu.sync_copy(zbuf, acc2.at[pl.ds(base, ECH)])
    plsc.subcore_barrier()

    _stage(table_hbm, idx1_hbm, acc1, iring, rows, isem, gsem, ssem, c, s)
    plsc.subcore_barrier()

    _normalize(acc1, rows, s,
               lambda base, eb: pltpu.sync_copy(eb, acc1.at[pl.ds(base, ECH)]))
    plsc.subcore_barrier()

    _stage(acc1, idx2_hbm, acc2, iring, rows, isem, gsem, ssem, c, s)
    plsc.subcore_barrier()

    _normalize(acc2, rows, s,
               lambda base, eb: pltpu.sync_copy(
                   eb, out_hbm.at[c, pl.ds(base, ECH)]))


@functools.partial(jax.jit, static_argnames=())
def _sc_conv(table, idx1, idx2, zrows):
    mesh = plsc.VectorSubcoreMesh(core_axis_name="c", subcore_axis_name="s",
                                  num_cores=NC, num_subcores=NS)
    return pl.kernel(
        _sc_body,
        out_type=jax.ShapeDtypeStruct((NC, N_NODES, W), jnp.float32),
        mesh=mesh,
        scratch_types=[
            pltpu.VMEM_SHARED((ACC_ROWS, W), jnp.float32),
            pltpu.VMEM_SHARED((ACC_ROWS, W), jnp.float32),
            pltpu.VMEM((RING, 2, K), jnp.int32),
            pltpu.VMEM((RING, K, W), jnp.float32),
            pltpu.SemaphoreType.DMA((RING,)),
            pltpu.SemaphoreType.DMA((RING,)),
            pltpu.SemaphoreType.DMA((RING,)),
        ],
        compiler_params=pltpu.CompilerParams(use_tc_tiling_on_sc=False),
    )(table, idx1, idx2, zrows)


def _table_body(x_ref, w_ref, o_ref):
    xwh = jnp.dot(x_ref[...], w_ref[0], preferred_element_type=jnp.float32)
    col = lax.broadcasted_iota(jnp.int32, (N_NODES, W - H), 1)
    aux = jnp.where(col == 0, jnp.float32(1.0), jnp.float32(0.0))
    o_ref[...] = jnp.concatenate([xwh, aux], axis=1)


def _mlp_body(s2_ref, x_ref, bc_ref, wm_ref, bm_ref, g_ref, b_ref, o_ref):
    h = (jnp.dot(s2_ref[0, :, :H], wm_ref[:H, :],
                 preferred_element_type=jnp.float32)
         + jnp.dot(s2_ref[1, :, :H], wm_ref[H:, :],
                   preferred_element_type=jnp.float32)
         + jnp.dot(bc_ref[...], wm_ref[...],
                   preferred_element_type=jnp.float32))
    h = h + bm_ref[...]
    mean = jnp.mean(h, axis=0, keepdims=True)
    var = jnp.mean((h - mean) ** 2, axis=0, keepdims=True)
    h = (h - mean) * lax.rsqrt(var + 1e-5)
    h = g_ref[...] * h + b_ref[...]
    h = jnp.where(h >= 0, h, 0.01 * h)
    r = h + x_ref[...]
    o_ref[...] = jnp.where(r >= 0, r, 0.01 * r)


def kernel(x, hyperedge_index, W_conv, b_conv, W_mlp, b_mlp, gamma, beta):
    node_idx = hyperedge_index[0].astype(jnp.int32)
    edge_idx = hyperedge_index[1].astype(jnp.int32)

    table1 = pl.pallas_call(
        _table_body,
        grid=(NC,),
        in_specs=[
            pl.BlockSpec((N_NODES, D), lambda c: (0, 0)),
            pl.BlockSpec((1, D, H), lambda c: (c, 0, 0)),
        ],
        out_specs=pl.BlockSpec((N_NODES, W), lambda c: (c, 0)),
        out_shape=jax.ShapeDtypeStruct((NC * N_NODES, W), jnp.float32),
    )(x, jnp.stack([W_conv[:, :H], W_conv[:, H:]]))

    n2 = node_idx.reshape(NS, INC_PER_TILE)
    e2 = edge_idx.reshape(NS, INC_PER_TILE)
    padg = jnp.zeros((NS, PAD), jnp.int32)
    pads = jnp.full((NS, PAD), TRASH, jnp.int32)
    n_g = jnp.concatenate([n2, padg], axis=1).reshape(NS, NB, K)
    n_s = jnp.concatenate([n2, pads], axis=1).reshape(NS, NB, K)
    e_g = jnp.concatenate([e2, padg], axis=1).reshape(NS, NB, K)
    e_s = jnp.concatenate([e2, pads], axis=1).reshape(NS, NB, K)
    nadj = jnp.stack([n_g, n_g + N_NODES])
    comb1 = jnp.stack(
        [nadj, jnp.broadcast_to(e_s, (NC, NS, NB, K))], axis=3)
    comb2 = jnp.broadcast_to(
        jnp.stack([e_g, n_s], axis=2), (NC, NS, NB, 2, K))
    zrows = jnp.zeros((ECH, W), jnp.float32)

    s2 = _sc_conv(table1, comb1, comb2, zrows)

    return pl.pallas_call(
        _mlp_body,
        out_shape=jax.ShapeDtypeStruct((N_NODES, D), jnp.float32),
    )(s2, x, b_conv.reshape(1, D), W_mlp, b_mlp.reshape(1, D),
      gamma.reshape(1, D), beta.reshape(1, D))


# speedup vs baseline: 1.1163x; 1.1163x over previous
import functools

import jax
import jax.numpy as jnp
from jax import lax
from jax.experimental import pallas as pl
from jax.experimental.pallas import tpu as pltpu
from jax.experimental.pallas import tpu_sc as plsc

N_NODES = 10000
N_EDGES = 10000
N_INC = 320000
D = 128
H = 64
W = 80
NC = 2
NS = 16
K = 128
INC_PER_TILE = N_INC // NS
NB = -(-INC_PER_TILE // K)
TAIL = INC_PER_TILE - (NB - 1) * K
TRASH = N_NODES
ACC_ROWS = N_NODES + 8
RING = 6
ROWS_PER_TILE = N_NODES // NS
ECH = 125


def _make_sc_body(g_row, s_row):
    def body(table_hbm, hei_hbm, zrows_hbm, out_hbm,
             acc_shared, iring, rows, ebuf, isem, gsem, ssem):
        c = lax.axis_index("c")
        s = lax.axis_index("s")
        goff = c * N_NODES

        pltpu.sync_copy(zrows_hbm, ebuf)
        for ch in range(ROWS_PER_TILE // ECH):
            pltpu.sync_copy(
                ebuf, acc_shared.at[pl.ds(s * ROWS_PER_TILE + ch * ECH, ECH)])
        plsc.subcore_barrier()

        tile_base = s * INC_PER_TILE

        def fire_idx(j):
            slot = lax.rem(j, RING)
            is_tail = j == NB - 1

            @pl.when(jnp.logical_not(is_tail))
            def _():
                pltpu.async_copy(hei_hbm.at[:, pl.ds(tile_base + j * K, K)],
                                 iring.at[slot], isem.at[slot])

            @pl.when(is_tail)
            def _():
                pltpu.async_copy(
                    hei_hbm.at[:, pl.ds(tile_base + j * K, TAIL)],
                    iring.at[slot, :, pl.ds(0, TAIL)], isem.at[slot])
                for q in range(TAIL // 16, K // 16):
                    iring[slot, g_row, pl.ds(q * 16, 16)] = jnp.zeros(
                        (16,), jnp.int32)
                    iring[slot, s_row, pl.ds(q * 16, 16)] = jnp.full(
                        (16,), TRASH, jnp.int32)

        def wait_idx_and_prep(j):
            slot = lax.rem(j, RING)
            is_tail = j == NB - 1

            @pl.when(jnp.logical_not(is_tail))
            def _():
                pltpu.make_async_copy(
                    hei_hbm.at[:, pl.ds(tile_base + j * K, K)],
                    iring.at[slot], isem.at[slot]).wait()

            @pl.when(is_tail)
            def _():
                pltpu.make_async_copy(
                    hei_hbm.at[:, pl.ds(tile_base + j * K, TAIL)],
                    iring.at[slot, :, pl.ds(0, TAIL)], isem.at[slot]).wait()

            for q in range(K // 16):
                iring[slot, g_row, pl.ds(q * 16, 16)] = (
                    iring[slot, g_row, pl.ds(q * 16, 16)] + goff)

        def fire_gather(j):
            slot = lax.rem(j, RING)
            pltpu.async_copy(table_hbm.at[iring.at[slot, g_row]],
                             rows.at[slot], gsem.at[slot])

        def wait_gather(j):
            slot = lax.rem(j, RING)
            pltpu.make_async_copy(table_hbm.at[iring.at[slot, g_row]],
                                  rows.at[slot], gsem.at[slot]).wait()

        def fire_scatter(j):
            slot = lax.rem(j, RING)
            pltpu.async_copy(rows.at[slot], acc_shared.at[iring.at[slot, s_row]],
                             ssem.at[slot], add=True)

        def wait_scatter(j):
            slot = lax.rem(j, RING)
            pltpu.make_async_copy(rows.at[slot],
                                  acc_shared.at[iring.at[slot, s_row]],
                                  ssem.at[slot]).wait()

        for t in range(RING):
            fire_idx(t)
        for g in range(2):
            wait_idx_and_prep(g)
            fire_gather(g)

        def step(j, _):
            @pl.when(jnp.logical_and(j >= 1, j - 1 + RING < NB))
            def _():
                wait_scatter(j - 1)
                fire_idx(j - 1 + RING)

            @pl.when(j + 2 < NB)
            def _():
                wait_idx_and_prep(j + 2)
                fire_gather(j + 2)

            wait_gather(j)
            fire_scatter(j)
            return 0

        lax.fori_loop(0, NB, step, 0)

        def drain(r, _):
            wait_scatter(r)
            return 0

        lax.fori_loop(NB - RING, NB, drain, 0)
        plsc.subcore_barrier()

        ones_first = jnp.where(lax.iota(jnp.int32, 16) == 0,
                               jnp.float32(1.0), jnp.float32(0.0))
        for ch in range(ROWS_PER_TILE // ECH):
            base = s * ROWS_PER_TILE + ch * ECH
            pltpu.sync_copy(acc_shared.at[pl.ds(base, ECH)], ebuf)

            def erow(i, _):
                cnt = ebuf[i, pl.ds(H, 16)][0]
                cntv = jnp.full((16,), cnt, jnp.float32)
                invv = jnp.where(cntv > 0.0, 1.0 / cntv, jnp.float32(0.0))
                for q in range(H // 16):
                    ebuf[i, pl.ds(q * 16, 16)] = (
                        ebuf[i, pl.ds(q * 16, 16)] * invv)
                ebuf[i, pl.ds(H, 16)] = ones_first
                return 0

            lax.fori_loop(0, ECH, erow, 0)
            pltpu.sync_copy(ebuf, out_hbm.at[c, pl.ds(base, ECH)])

    return body


@functools.partial(jax.jit, static_argnames=("g_row", "s_row"))
def _sc_pass(table, hei, zrows, *, g_row, s_row):
    mesh = plsc.VectorSubcoreMesh(core_axis_name="c", subcore_axis_name="s",
                                  num_cores=NC, num_subcores=NS)
    return pl.kernel(
        _make_sc_body(g_row, s_row),
        out_type=jax.ShapeDtypeStruct((NC, N_NODES, W), jnp.float32),
        mesh=mesh,
        scratch_types=[
            pltpu.VMEM_SHARED((ACC_ROWS, W), jnp.float32),
            pltpu.VMEM((RING, 2, K), jnp.int32),
            pltpu.VMEM((RING, K, W), jnp.float32),
            pltpu.VMEM((ECH, W), jnp.float32),
            pltpu.SemaphoreType.DMA((RING,)),
            pltpu.SemaphoreType.DMA((RING,)),
            pltpu.SemaphoreType.DMA((RING,)),
        ],
        compiler_params=pltpu.CompilerParams(use_tc_tiling_on_sc=False),
    )(table, hei, zrows)


def _table_body(x_ref, w_ref, o_ref):
    xwh = jnp.dot(x_ref[...], w_ref[0], preferred_element_type=jnp.float32)
    col = lax.broadcasted_iota(jnp.int32, (N_NODES, W - H), 1)
    aux = jnp.where(col == 0, jnp.float32(1.0), jnp.float32(0.0))
    o_ref[...] = jnp.concatenate([xwh, aux], axis=1)


def _mlp_body(s2_ref, x_ref, bc_ref, wm_ref, bm_ref, g_ref, b_ref, o_ref):
    h = (jnp.dot(s2_ref[0, :, :H], wm_ref[:H, :],
                 preferred_element_type=jnp.float32)
         + jnp.dot(s2_ref[1, :, :H], wm_ref[H:, :],
                   preferred_element_type=jnp.float32)
         + jnp.dot(bc_ref[...], wm_ref[...],
                   preferred_element_type=jnp.float32))
    h = h + bm_ref[...]
    mean = jnp.mean(h, axis=0, keepdims=True)
    var = jnp.mean((h - mean) ** 2, axis=0, keepdims=True)
    h = (h - mean) * lax.rsqrt(var + 1e-5)
    h = g_ref[...] * h + b_ref[...]
    h = jnp.where(h >= 0, h, 0.01 * h)
    r = h + x_ref[...]
    o_ref[...] = jnp.where(r >= 0, r, 0.01 * r)


def kernel(x, hyperedge_index, W_conv, b_conv, W_mlp, b_mlp, gamma, beta):
    hei = hyperedge_index.astype(jnp.int32)

    table1 = pl.pallas_call(
        _table_body,
        grid=(NC,),
        in_specs=[
            pl.BlockSpec((N_NODES, D), lambda c: (0, 0)),
            pl.BlockSpec((1, D, H), lambda c: (c, 0, 0)),
        ],
        out_specs=pl.BlockSpec((N_NODES, W), lambda c: (c, 0)),
        out_shape=jax.ShapeDtypeStruct((NC * N_NODES, W), jnp.float32),
    )(x, jnp.stack([W_conv[:, :H], W_conv[:, H:]]))

    zrows = jnp.zeros((ECH, W), jnp.float32)

    s1 = _sc_pass(table1, hei, zrows, g_row=0, s_row=1)
    s2 = _sc_pass(s1.reshape(NC * N_NODES, W), hei, zrows, g_row=1, s_row=0)

    return pl.pallas_call(
        _mlp_body,
        out_shape=jax.ShapeDtypeStruct((N_NODES, D), jnp.float32),
    )(s2, x, b_conv.reshape(1, D), W_mlp, b_mlp.reshape(1, D),
      gamma.reshape(1, D), beta.reshape(1, D))


# speedup vs baseline: 1.1611x; 1.0401x over previous
import functools

import jax
import jax.numpy as jnp
from jax import lax
from jax.experimental import pallas as pl
from jax.experimental.pallas import tpu as pltpu
from jax.experimental.pallas import tpu_sc as plsc

N_NODES = 10000
N_EDGES = 10000
N_INC = 320000
D = 128
H = 64
W = 72
NC = 2
NS = 16
K = 128
INC_PER_TILE = N_INC // NS
NB = -(-INC_PER_TILE // K)
TAIL = INC_PER_TILE - (NB - 1) * K
TRASH = N_NODES
ACC_ROWS = N_NODES + 8
RING = 6
ROWS_PER_TILE = N_NODES // NS
ECH = 125


def _make_sc_body(g_row, s_row):
    def body(table_hbm, hei_hbm, zrows_hbm, out_hbm,
             acc_shared, iring, rows, ebuf, isem, gsem, ssem):
        c = lax.axis_index("c")
        s = lax.axis_index("s")
        goff = c * N_NODES

        pltpu.sync_copy(zrows_hbm, ebuf)
        for ch in range(ROWS_PER_TILE // ECH):
            pltpu.sync_copy(
                ebuf, acc_shared.at[pl.ds(s * ROWS_PER_TILE + ch * ECH, ECH)])
        plsc.subcore_barrier()

        tile_base = s * INC_PER_TILE

        def fire_idx(j):
            slot = lax.rem(j, RING)
            is_tail = j == NB - 1

            @pl.when(jnp.logical_not(is_tail))
            def _():
                pltpu.async_copy(hei_hbm.at[:, pl.ds(tile_base + j * K, K)],
                                 iring.at[slot], isem.at[slot])

            @pl.when(is_tail)
            def _():
                pltpu.async_copy(
                    hei_hbm.at[:, pl.ds(tile_base + j * K, TAIL)],
                    iring.at[slot, :, pl.ds(0, TAIL)], isem.at[slot])
                for q in range(TAIL // 16, K // 16):
                    iring[slot, g_row, pl.ds(q * 16, 16)] = jnp.zeros(
                        (16,), jnp.int32)
                    iring[slot, s_row, pl.ds(q * 16, 16)] = jnp.full(
                        (16,), TRASH, jnp.int32)

        def wait_idx_and_prep(j):
            slot = lax.rem(j, RING)
            is_tail = j == NB - 1

            @pl.when(jnp.logical_not(is_tail))
            def _():
                pltpu.make_async_copy(
                    hei_hbm.at[:, pl.ds(tile_base + j * K, K)],
                    iring.at[slot], isem.at[slot]).wait()

            @pl.when(is_tail)
            def _():
                pltpu.make_async_copy(
                    hei_hbm.at[:, pl.ds(tile_base + j * K, TAIL)],
                    iring.at[slot, :, pl.ds(0, TAIL)], isem.at[slot]).wait()

            for q in range(K // 16):
                iring[slot, g_row, pl.ds(q * 16, 16)] = (
                    iring[slot, g_row, pl.ds(q * 16, 16)] + goff)

        def fire_gather(j):
            slot = lax.rem(j, RING)
            pltpu.async_copy(table_hbm.at[iring.at[slot, g_row]],
                             rows.at[slot], gsem.at[slot])

        def wait_gather(j):
            slot = lax.rem(j, RING)
            pltpu.make_async_copy(table_hbm.at[iring.at[slot, g_row]],
                                  rows.at[slot], gsem.at[slot]).wait()

        def fire_scatter(j):
            slot = lax.rem(j, RING)
            pltpu.async_copy(rows.at[slot], acc_shared.at[iring.at[slot, s_row]],
                             ssem.at[slot], add=True)

        def wait_scatter(j):
            slot = lax.rem(j, RING)
            pltpu.make_async_copy(rows.at[slot],
                                  acc_shared.at[iring.at[slot, s_row]],
                                  ssem.at[slot]).wait()

        for t in range(RING):
            fire_idx(t)
        for g in range(2):
            wait_idx_and_prep(g)
            fire_gather(g)

        def step(j, _):
            @pl.when(jnp.logical_and(j >= 1, j - 1 + RING < NB))
            def _():
                wait_scatter(j - 1)
                fire_idx(j - 1 + RING)

            @pl.when(j + 2 < NB)
            def _():
                wait_idx_and_prep(j + 2)
                fire_gather(j + 2)

            wait_gather(j)
            fire_scatter(j)
            return 0

        lax.fori_loop(0, NB, step, 0)

        def drain(r, _):
            wait_scatter(r)
            return 0

        lax.fori_loop(NB - RING, NB, drain, 0)
        plsc.subcore_barrier()

        lane = lax.iota(jnp.int32, 16)
        for ch in range(ROWS_PER_TILE // ECH):
            base = s * ROWS_PER_TILE + ch * ECH
            pltpu.sync_copy(acc_shared.at[pl.ds(base, ECH)], ebuf)

            def erow(i, _):
                tailv = ebuf[i, pl.ds(W - 16, 16)]
                cnt = tailv[8]
                cntv = jnp.full((16,), cnt, jnp.float32)
                invv = jnp.where(cntv > 0.0, 1.0 / cntv, jnp.float32(0.0))
                tail = jnp.where(lane < 8, tailv * invv,
                                 jnp.where(lane == 8, jnp.float32(1.0),
                                           jnp.float32(0.0)))
                for q in range(H // 16):
                    ebuf[i, pl.ds(q * 16, 16)] = (
                        ebuf[i, pl.ds(q * 16, 16)] * invv)
                ebuf[i, pl.ds(W - 16, 16)] = tail
                return 0

            lax.fori_loop(0, ECH, erow, 0)
            pltpu.sync_copy(ebuf, out_hbm.at[c, pl.ds(base, ECH)])

    return body


@functools.partial(jax.jit, static_argnames=("g_row", "s_row"))
def _sc_pass(table, hei, zrows, *, g_row, s_row):
    mesh = plsc.VectorSubcoreMesh(core_axis_name="c", subcore_axis_name="s",
                                  num_cores=NC, num_subcores=NS)
    return pl.kernel(
        _make_sc_body(g_row, s_row),
        out_type=jax.ShapeDtypeStruct((NC, N_NODES, W), jnp.float32),
        mesh=mesh,
        scratch_types=[
            pltpu.VMEM_SHARED((ACC_ROWS, W), jnp.float32),
            pltpu.VMEM((RING, 2, K), jnp.int32),
            pltpu.VMEM((RING, K, W), jnp.float32),
            pltpu.VMEM((ECH, W), jnp.float32),
            pltpu.SemaphoreType.DMA((RING,)),
            pltpu.SemaphoreType.DMA((RING,)),
            pltpu.SemaphoreType.DMA((RING,)),
        ],
        compiler_params=pltpu.CompilerParams(use_tc_tiling_on_sc=False),
    )(table, hei, zrows)


def _table_body(x_ref, w_ref, o_ref):
    xwh = jnp.dot(x_ref[...], w_ref[0], preferred_element_type=jnp.float32)
    col = lax.broadcasted_iota(jnp.int32, (N_NODES, W - H), 1)
    aux = jnp.where(col == 0, jnp.float32(1.0), jnp.float32(0.0))
    o_ref[...] = jnp.concatenate([xwh, aux], axis=1)


def _mlp_body(s2_ref, x_ref, bc_ref, wm_ref, bm_ref, g_ref, b_ref, o_ref):
    h = (jnp.dot(s2_ref[0, :, :H], wm_ref[:H, :],
                 preferred_element_type=jnp.float32)
         + jnp.dot(s2_ref[1, :, :H], wm_ref[H:, :],
                   preferred_element_type=jnp.float32)
         + jnp.dot(bc_ref[...], wm_ref[...],
                   preferred_element_type=jnp.float32))
    h = h + bm_ref[...]
    mean = jnp.mean(h, axis=0, keepdims=True)
    var = jnp.mean((h - mean) ** 2, axis=0, keepdims=True)
    h = (h - mean) * lax.rsqrt(var + 1e-5)
    h = g_ref[...] * h + b_ref[...]
    h = jnp.where(h >= 0, h, 0.01 * h)
    r = h + x_ref[...]
    o_ref[...] = jnp.where(r >= 0, r, 0.01 * r)


def kernel(x, hyperedge_index, W_conv, b_conv, W_mlp, b_mlp, gamma, beta):
    hei = hyperedge_index.astype(jnp.int32)

    table1 = pl.pallas_call(
        _table_body,
        grid=(NC,),
        in_specs=[
            pl.BlockSpec((N_NODES, D), lambda c: (0, 0)),
            pl.BlockSpec((1, D, H), lambda c: (c, 0, 0)),
        ],
        out_specs=pl.BlockSpec((N_NODES, W), lambda c: (c, 0)),
        out_shape=jax.ShapeDtypeStruct((NC * N_NODES, W), jnp.float32),
    )(x, jnp.stack([W_conv[:, :H], W_conv[:, H:]]))

    zrows = jnp.zeros((ECH, W), jnp.float32)

    s1 = _sc_pass(table1, hei, zrows, g_row=0, s_row=1)
    s2 = _sc_pass(s1.reshape(NC * N_NODES, W), hei, zrows, g_row=1, s_row=0)

    return pl.pallas_call(
        _mlp_body,
        out_shape=jax.ShapeDtypeStruct((N_NODES, D), jnp.float32),
    )(s2, x, b_conv.reshape(1, D), W_mlp, b_mlp.reshape(1, D),
      gamma.reshape(1, D), beta.reshape(1, D))


# speedup vs baseline: 1.2077x; 1.0401x over previous
import functools

import jax
import jax.numpy as jnp
from jax import lax
from jax.experimental import pallas as pl
from jax.experimental.pallas import tpu as pltpu
from jax.experimental.pallas import tpu_sc as plsc

N_NODES = 10000
N_EDGES = 10000
N_INC = 320000
D = 128
H = 64
W = 64
CW = 16
NC = 2
NS = 16
K = 128
INC_PER_TILE = N_INC // NS
NB = -(-INC_PER_TILE // K)
TAIL = INC_PER_TILE - (NB - 1) * K
TRASH = N_NODES
ACC_ROWS = N_NODES + 8
RING = 4
ROWS_PER_TILE = N_NODES // NS
ECH = 125


def _make_sc_body(g_row, s_row):
    def body(table_hbm, hei_hbm, zrows_hbm, out_hbm,
             acc_shared, cacc_shared, iring, rows, ones, ebuf, cbuf,
             isem, gsem, ssem, csem):
        c = lax.axis_index("c")
        s = lax.axis_index("s")
        goff = c * N_NODES

        def fill(i, _):
            ones[i, pl.ds(0, CW)] = jnp.ones((16,), jnp.float32)
            return 0

        lax.fori_loop(0, K, fill, 0)

        def zfill(i, _):
            cbuf[i, pl.ds(0, CW)] = jnp.zeros((16,), jnp.float32)
            return 0

        lax.fori_loop(0, ECH, zfill, 0)

        pltpu.sync_copy(zrows_hbm, ebuf)
        for ch in range(ROWS_PER_TILE // ECH):
            base = s * ROWS_PER_TILE + ch * ECH
            pltpu.sync_copy(ebuf, acc_shared.at[pl.ds(base, ECH)])
            pltpu.sync_copy(cbuf, cacc_shared.at[pl.ds(base, ECH)])
        plsc.subcore_barrier()

        tile_base = s * INC_PER_TILE

        def fire_idx(j):
            slot = lax.rem(j, RING)
            is_tail = j == NB - 1

            @pl.when(jnp.logical_not(is_tail))
            def _():
                pltpu.async_copy(hei_hbm.at[:, pl.ds(tile_base + j * K, K)],
                                 iring.at[slot], isem.at[slot])

            @pl.when(is_tail)
            def _():
                pltpu.async_copy(
                    hei_hbm.at[:, pl.ds(tile_base + j * K, TAIL)],
                    iring.at[slot, :, pl.ds(0, TAIL)], isem.at[slot])
                for q in range(TAIL // 16, K // 16):
                    iring[slot, g_row, pl.ds(q * 16, 16)] = jnp.zeros(
                        (16,), jnp.int32)
                    iring[slot, s_row, pl.ds(q * 16, 16)] = jnp.full(
                        (16,), TRASH, jnp.int32)

        def wait_idx_and_prep(j):
            slot = lax.rem(j, RING)
            is_tail = j == NB - 1

            @pl.when(jnp.logical_not(is_tail))
            def _():
                pltpu.make_async_copy(
                    hei_hbm.at[:, pl.ds(tile_base + j * K, K)],
                    iring.at[slot], isem.at[slot]).wait()

            @pl.when(is_tail)
            def _():
                pltpu.make_async_copy(
                    hei_hbm.at[:, pl.ds(tile_base + j * K, TAIL)],
                    iring.at[slot, :, pl.ds(0, TAIL)], isem.at[slot]).wait()

            for q in range(K // 16):
                iring[slot, g_row, pl.ds(q * 16, 16)] = (
                    iring[slot, g_row, pl.ds(q * 16, 16)] + goff)

        def fire_gather(j):
            slot = lax.rem(j, RING)
            pltpu.async_copy(table_hbm.at[iring.at[slot, g_row]],
                             rows.at[slot], gsem.at[slot])

        def wait_gather(j):
            slot = lax.rem(j, RING)
            pltpu.make_async_copy(table_hbm.at[iring.at[slot, g_row]],
                                  rows.at[slot], gsem.at[slot]).wait()

        def fire_scatter(j):
            slot = lax.rem(j, RING)
            pltpu.async_copy(rows.at[slot], acc_shared.at[iring.at[slot, s_row]],
                             ssem.at[slot], add=True)
            pltpu.async_copy(ones, cacc_shared.at[iring.at[slot, s_row]],
                             csem.at[slot], add=True)

        def wait_scatter(j):
            slot = lax.rem(j, RING)
            pltpu.make_async_copy(rows.at[slot],
                                  acc_shared.at[iring.at[slot, s_row]],
                                  ssem.at[slot]).wait()
            pltpu.make_async_copy(ones, cacc_shared.at[iring.at[slot, s_row]],
                                  csem.at[slot]).wait()

        for t in range(RING):
            fire_idx(t)
        for g in range(2):
            wait_idx_and_prep(g)
            fire_gather(g)

        def step(j, _):
            @pl.when(jnp.logical_and(j >= 1, j - 1 + RING < NB))
            def _():
                wait_scatter(j - 1)
                fire_idx(j - 1 + RING)

            @pl.when(j + 2 < NB)
            def _():
                wait_idx_and_prep(j + 2)
                fire_gather(j + 2)

            wait_gather(j)
            fire_scatter(j)
            return 0

        lax.fori_loop(0, NB, step, 0)

        def drain(r, _):
            wait_scatter(r)
            return 0

        lax.fori_loop(NB - RING, NB, drain, 0)
        plsc.subcore_barrier()

        for ch in range(ROWS_PER_TILE // ECH):
            base = s * ROWS_PER_TILE + ch * ECH
            pltpu.sync_copy(acc_shared.at[pl.ds(base, ECH)], ebuf)
            pltpu.sync_copy(cacc_shared.at[pl.ds(base, ECH)], cbuf)

            def erow(i, _):
                cntv = cbuf[i, pl.ds(0, 16)]
                invv = jnp.where(cntv > 0.0, 1.0 / cntv, jnp.float32(0.0))
                for q in range(W // 16):
                    ebuf[i, pl.ds(q * 16, 16)] = (
                        ebuf[i, pl.ds(q * 16, 16)] * invv)
                return 0

            lax.fori_loop(0, ECH, erow, 0)
            pltpu.sync_copy(ebuf, out_hbm.at[c, pl.ds(base, ECH)])

    return body


@functools.partial(jax.jit, static_argnames=("g_row", "s_row"))
def _sc_pass(table, hei, zrows, *, g_row, s_row):
    mesh = plsc.VectorSubcoreMesh(core_axis_name="c", subcore_axis_name="s",
                                  num_cores=NC, num_subcores=NS)
    return pl.kernel(
        _make_sc_body(g_row, s_row),
        out_type=jax.ShapeDtypeStruct((NC, N_NODES, W), jnp.float32),
        mesh=mesh,
        scratch_types=[
            pltpu.VMEM_SHARED((ACC_ROWS, W), jnp.float32),
            pltpu.VMEM_SHARED((ACC_ROWS, CW), jnp.float32),
            pltpu.VMEM((RING, 2, K), jnp.int32),
            pltpu.VMEM((RING, K, W), jnp.float32),
            pltpu.VMEM((K, CW), jnp.float32),
            pltpu.VMEM((ECH, W), jnp.float32),
            pltpu.VMEM((ECH, CW), jnp.float32),
            pltpu.SemaphoreType.DMA((RING,)),
            pltpu.SemaphoreType.DMA((RING,)),
            pltpu.SemaphoreType.DMA((RING,)),
            pltpu.SemaphoreType.DMA((RING,)),
        ],
        compiler_params=pltpu.CompilerParams(use_tc_tiling_on_sc=False),
    )(table, hei, zrows)


def _table_body(x_ref, w_ref, o_ref):
    o_ref[...] = jnp.dot(x_ref[...], w_ref[0],
                         preferred_element_type=jnp.float32)


def _mlp_body(s2_ref, x_ref, bc_ref, wm_ref, bm_ref, g_ref, b_ref, o_ref):
    h = (jnp.dot(s2_ref[0], wm_ref[:H, :],
                 preferred_element_type=jnp.float32)
         + jnp.dot(s2_ref[1], wm_ref[H:, :],
                   preferred_element_type=jnp.float32)
         + jnp.dot(bc_ref[...], wm_ref[...],
                   preferred_element_type=jnp.float32))
    h = h + bm_ref[...]
    mean = jnp.mean(h, axis=0, keepdims=True)
    var = jnp.mean((h - mean) ** 2, axis=0, keepdims=True)
    h = (h - mean) * lax.rsqrt(var + 1e-5)
    h = g_ref[...] * h + b_ref[...]
    h = jnp.where(h >= 0, h, 0.01 * h)
    r = h + x_ref[...]
    o_ref[...] = jnp.where(r >= 0, r, 0.01 * r)


def kernel(x, hyperedge_index, W_conv, b_conv, W_mlp, b_mlp, gamma, beta):
    hei = hyperedge_index.astype(jnp.int32)

    table1 = pl.pallas_call(
        _table_body,
        grid=(NC,),
        in_specs=[
            pl.BlockSpec((N_NODES, D), lambda c: (0, 0)),
            pl.BlockSpec((1, D, H), lambda c: (c, 0, 0)),
        ],
        out_specs=pl.BlockSpec((N_NODES, W), lambda c: (c, 0)),
        out_shape=jax.ShapeDtypeStruct((NC * N_NODES, W), jnp.float32),
    )(x, jnp.stack([W_conv[:, :H], W_conv[:, H:]]))

    zrows = jnp.zeros((ECH, W), jnp.float32)

    s1 = _sc_pass(table1, hei, zrows, g_row=0, s_row=1)
    s2 = _sc_pass(s1.reshape(NC * N_NODES, W), hei, zrows, g_row=1, s_row=0)

    return pl.pallas_call(
        _mlp_body,
        out_shape=jax.ShapeDtypeStruct((N_NODES, D), jnp.float32),
    )(s2, x, b_conv.reshape(1, D), W_mlp, b_mlp.reshape(1, D),
      gamma.reshape(1, D), beta.reshape(1, D))
